# Initial kernel scaffold; baseline (speedup 1.0000x reference)
import jax
import jax.numpy as jnp
from jax.experimental import pallas as pl


def kernel(x, edge_index, edge_type, batch, W_in, b_in, W_rel, W_root, b_conv, ln_g, ln_b):
    raise NotImplementedError("write your pallas kernel here")


# speedup vs baseline: 14.7067x; 14.7067x over previous
import functools

import jax
import jax.numpy as jnp
from jax import lax
from jax.experimental import pallas as pl
from jax.experimental.pallas import tpu as pltpu
from jax.experimental.pallas import tpu_sc as plsc


_BLK = 1000


def _proj_body(x_ref, w_ref, b_ref, o_ref):
    o_ref[...] = (
        jnp.dot(x_ref[...], w_ref[...], preferred_element_type=jnp.float32)
        + b_ref[...]
    )


def _tc_proj(x, w, b):
    n, d = x.shape
    h = w.shape[1]
    return pl.pallas_call(
        _proj_body,
        grid=(n // _BLK,),
        in_specs=[
            pl.BlockSpec((_BLK, d), lambda i: (i, 0)),
            pl.BlockSpec((d, h), lambda i: (0, 0)),
            pl.BlockSpec((1, h), lambda i: (0, 0)),
        ],
        out_specs=pl.BlockSpec((_BLK, h), lambda i: (i, 0)),
        out_shape=jax.ShapeDtypeStruct((n, h), jnp.float32),
    )(x, w, b.reshape(1, h))


def _layer_a_body(h_ref, wrel_ref, wroot_ref, b_ref, hall_ref, root_ref):
    hv = h_ref[...]
    r = wrel_ref.shape[0]
    for i in range(r):
        hall_ref[i] = jnp.dot(hv, wrel_ref[i], preferred_element_type=jnp.float32)
    root_ref[...] = (
        jnp.dot(hv, wroot_ref[...], preferred_element_type=jnp.float32) + b_ref[...]
    )


def _tc_layer_a(hx, wrel, wroot, b):
    n, h = hx.shape
    r = wrel.shape[0]
    return pl.pallas_call(
        _layer_a_body,
        grid=(n // _BLK,),
        in_specs=[
            pl.BlockSpec((_BLK, h), lambda i: (i, 0)),
            pl.BlockSpec((r, h, h), lambda i: (0, 0, 0)),
            pl.BlockSpec((h, h), lambda i: (0, 0)),
            pl.BlockSpec((1, h), lambda i: (0, 0)),
        ],
        out_specs=[
            pl.BlockSpec((r, _BLK, h), lambda i: (0, i, 0)),
            pl.BlockSpec((_BLK, h), lambda i: (i, 0)),
        ],
        out_shape=[
            jax.ShapeDtypeStruct((r, n, h), jnp.float32),
            jax.ShapeDtypeStruct((n, h), jnp.float32),
        ],
    )(hx, wrel, wroot, b.reshape(1, h))


def _layer_c_body(agg_ref, root_ref, h_ref, g_ref, b_ref, o_ref):
    s = agg_ref[0] + agg_ref[1] + root_ref[...]
    mu = jnp.mean(s, axis=-1, keepdims=True)
    var = jnp.mean((s - mu) ** 2, axis=-1, keepdims=True)
    y = (s - mu) / jnp.sqrt(var + 1e-5) * g_ref[...] + b_ref[...]
    o_ref[...] = jnp.maximum(y, 0.0) + h_ref[...]


def _tc_layer_c(agg2, root, hx, g, b):
    n, h = hx.shape
    return pl.pallas_call(
        _layer_c_body,
        grid=(n // _BLK,),
        in_specs=[
            pl.BlockSpec((2, _BLK, h), lambda i: (0, i, 0)),
            pl.BlockSpec((_BLK, h), lambda i: (i, 0)),
            pl.BlockSpec((_BLK, h), lambda i: (i, 0)),
            pl.BlockSpec((1, h), lambda i: (0, 0)),
            pl.BlockSpec((1, h), lambda i: (0, 0)),
        ],
        out_specs=pl.BlockSpec((_BLK, h), lambda i: (i, 0)),
        out_shape=jax.ShapeDtypeStruct((n, h), jnp.float32),
    )(agg2, root, hx, g.reshape(1, h), b.reshape(1, h))


def _inv_body(cnt_ref, o_ref):
    c = cnt_ref[0] + cnt_ref[1]
    o_ref[...] = jnp.where(c > 0, 1.0 / jnp.maximum(c, 1.0), 0.0)


def _tc_inv(cnt2_3d):
    _, rows, cols = cnt2_3d.shape
    return pl.pallas_call(
        _inv_body,
        grid=(1,),
        in_specs=[pl.BlockSpec((2, rows, cols), lambda i: (0, 0, 0))],
        out_specs=pl.BlockSpec((rows, cols), lambda i: (0, 0)),
        out_shape=jax.ShapeDtypeStruct((rows, cols), jnp.float32),
    )(cnt2_3d)


def _pool_body(b_ref, h_ref, o_ref):
    g = o_ref.shape[0]
    blk = h_ref.shape[0]

    @pl.when(pl.program_id(0) == 0)
    def _():
        o_ref[...] = jnp.zeros_like(o_ref)

    bvec = b_ref[0]
    onehot = (
        bvec == lax.broadcasted_iota(jnp.int32, (g, blk), 0)
    ).astype(jnp.float32)
    o_ref[...] += lax.dot_general(
        onehot,
        h_ref[...],
        (((1,), (0,)), ((), ())),
        preferred_element_type=jnp.float32,
    )


def _tc_pool(hx, batch3d, g):
    n, h = hx.shape
    grid = n // _BLK
    return pl.pallas_call(
        _pool_body,
        grid=(grid,),
        in_specs=[
            pl.BlockSpec((1, 1, _BLK), lambda i: (i, 0, 0)),
            pl.BlockSpec((_BLK, h), lambda i: (i, 0)),
        ],
        out_specs=pl.BlockSpec((g, h), lambda i: (0, 0)),
        out_shape=jax.ShapeDtypeStruct((g, h), jnp.float32),
    )(batch3d, hx)


_LANES = 16


def _zero_fill(ref, nelem):
    z = jnp.zeros((_LANES,), jnp.float32)

    def body(i, _):
        ref[pl.ds(i * _LANES, _LANES)] = z
        return 0

    lax.fori_loop(0, nelem // _LANES, body, 0, unroll=4)


def _sc_count(key16, nk, nc, ns):
    e = key16.shape[0]
    nw = nc * ns
    per_w = e // nw
    ck = 2000
    n_chunks = per_w // ck
    per_tile = nk // ns
    mesh = plsc.VectorSubcoreMesh(core_axis_name="c", subcore_axis_name="s")

    @functools.partial(
        pl.kernel,
        out_type=jax.ShapeDtypeStruct((2 * nk,), jnp.float32),
        mesh=mesh,
        scratch_types=[
            pltpu.VMEM((ck,), jnp.int32),
            pltpu.VMEM((ck,), jnp.float32),
            pltpu.VMEM((per_tile,), jnp.float32),
            pltpu.VMEM_SHARED((nk,), jnp.float32),
            pltpu.SemaphoreType.DMA,
        ],
    )
    def k(key_hbm, out_hbm, key_v, ones_v, zb, cnt_sp, sem):
        cid = lax.axis_index("c")
        sid = lax.axis_index("s")
        wid = sid * nc + cid

        o = jnp.ones((_LANES,), jnp.float32)

        def fill_ones(i, _):
            ones_v[pl.ds(i * _LANES, _LANES)] = o
            return 0

        lax.fori_loop(0, ck // _LANES, fill_ones, 0, unroll=4)

        _zero_fill(zb, per_tile)
        pltpu.sync_copy(zb, cnt_sp.at[pl.ds(sid * per_tile, per_tile)])
        plsc.subcore_barrier()

        def chunk(i, _):
            base = wid * per_w + i * ck
            pltpu.sync_copy(key_hbm.at[pl.ds(base, ck)], key_v)
            pltpu.sync_copy(ones_v, cnt_sp.at[key_v], add=True)
            return 0

        lax.fori_loop(0, n_chunks, chunk, 0)
        plsc.subcore_barrier()

        pltpu.sync_copy(cnt_sp.at[pl.ds(sid * per_tile, per_tile)], zb)
        pltpu.sync_copy(zb, out_hbm.at[pl.ds(cid * nk + sid * per_tile, per_tile)])

    return k(key16).reshape(2, nk)


def _sc_norm_gather(invflat, key16, nc, ns):
    e = key16.shape[0]
    nw = nc * ns
    per_w = e // nw
    ck = 2000
    n_chunks = per_w // ck
    mesh = plsc.VectorSubcoreMesh(core_axis_name="c", subcore_axis_name="s")

    @functools.partial(
        pl.kernel,
        out_type=jax.ShapeDtypeStruct((e,), jnp.float32),
        mesh=mesh,
        scratch_types=[
            pltpu.VMEM((ck,), jnp.int32),
            pltpu.VMEM((ck,), jnp.float32),
            pltpu.SemaphoreType.DMA,
        ],
    )
    def k(inv_hbm, key_hbm, out_hbm, key_v, nv, sem):
        cid = lax.axis_index("c")
        sid = lax.axis_index("s")
        wid = sid * nc + cid

        def chunk(i, _):
            base = wid * per_w + i * ck
            pltpu.sync_copy(key_hbm.at[pl.ds(base, ck)], key_v)
            pltpu.async_copy(inv_hbm.at[key_v], nv, sem).wait()
            pltpu.sync_copy(nv, out_hbm.at[pl.ds(base, ck)])
            return 0

        lax.fori_loop(0, n_chunks, chunk, 0)

    return k(invflat, key16)


def _sc_layer_agg(hall_flat, idx, dst, norm, n, h, nc, ns):
    e = idx.shape[0]
    nw = nc * ns
    per_w = e // nw
    ek = 80
    n_chunks = per_w // ek
    zrows = ek
    n_blocks = n // zrows
    mesh = plsc.VectorSubcoreMesh(core_axis_name="c", subcore_axis_name="s")

    @functools.partial(
        pl.kernel,
        out_type=jax.ShapeDtypeStruct((2, n, h), jnp.float32),
        mesh=mesh,
        scratch_types=[
            pltpu.VMEM((ek,), jnp.int32),
            pltpu.VMEM((ek,), jnp.int32),
            pltpu.VMEM((ek,), jnp.float32),
            pltpu.VMEM((ek, h), jnp.float32),
            pltpu.VMEM_SHARED((n, h), jnp.float32),
            pltpu.SemaphoreType.DMA,
        ],
    )
    def k(hall_hbm, idx_hbm, dst_hbm, norm_hbm, out_hbm,
          idx_v, dst_v, norm_v, rows_v, agg_sp, sem):
        cid = lax.axis_index("c")
        sid = lax.axis_index("s")
        wid = sid * nc + cid

        z = jnp.zeros((_LANES,), jnp.float32)

        def zfill(i, _):
            r = i // (h // _LANES)
            c = i % (h // _LANES)
            rows_v[r, pl.ds(c * _LANES, _LANES)] = z
            return 0

        lax.fori_loop(0, zrows * (h // _LANES), zfill, 0, unroll=4)
        zsrc = rows_v.at[pl.ds(0, zrows)]
        for j in range((n_blocks + ns - 1) // ns):
            blk = j * ns + sid

            @pl.when(blk < n_blocks)
            def _():
                pltpu.sync_copy(zsrc, agg_sp.at[pl.ds(blk * zrows, zrows)])
        plsc.subcore_barrier()

        def chunk(ci, _):
            base = wid * per_w + ci * ek
            pltpu.sync_copy(idx_hbm.at[pl.ds(base, ek)], idx_v)
            pltpu.sync_copy(dst_hbm.at[pl.ds(base, ek)], dst_v)
            pltpu.sync_copy(norm_hbm.at[pl.ds(base, ek)], norm_v)
            pltpu.async_copy(hall_hbm.at[idx_v], rows_v, sem).wait()

            def group(gi, _):
                nv16 = norm_v[pl.ds(gi * _LANES, _LANES)]
                for l in range(_LANES):
                    e = gi * _LANES + l
                    nb = jnp.broadcast_to(nv16[l], (_LANES,))
                    for c in range(h // _LANES):
                        sl = pl.ds(c * _LANES, _LANES)
                        rows_v[e, sl] = rows_v[e, sl] * nb
                return 0

            lax.fori_loop(0, ek // _LANES, group, 0)
            pltpu.sync_copy(rows_v, agg_sp.at[dst_v], add=True)
            return 0

        lax.fori_loop(0, n_chunks, chunk, 0)
        plsc.subcore_barrier()

        for j in range((n_blocks + ns - 1) // ns):
            blk = j * ns + sid

            @pl.when(blk < n_blocks)
            def _():
                r0 = blk * zrows
                pltpu.sync_copy(agg_sp.at[pl.ds(r0, zrows)], zsrc)
                pltpu.sync_copy(zsrc, out_hbm.at[cid, pl.ds(r0, zrows)])

    return k(hall_flat, idx, dst, norm)


def kernel(x, edge_index, edge_type, batch, W_in, b_in, W_rel, W_root,
           b_conv, ln_g, ln_b):
    n, d = x.shape
    hdim = W_in.shape[1]
    nlayers, r = W_rel.shape[0], W_rel.shape[1]
    g = 16
    info = plsc.get_sparse_core_info()
    nc, ns = info.num_cores, info.num_subcores

    src = edge_index[0]
    dst = edge_index[1]
    idx = edge_type * n + src
    key16 = dst * 16 + edge_type

    nk = n * 16
    cnt2 = _sc_count(key16, nk, nc, ns)
    inv = _tc_inv(cnt2.reshape(2, nk // 128, 128))
    norm = _sc_norm_gather(inv.reshape(nk), key16, nc, ns)

    h = _tc_proj(x, W_in, b_in)
    for l in range(nlayers):
        h_all, root = _tc_layer_a(h, W_rel[l], W_root[l], b_conv[l])
        agg2 = _sc_layer_agg(
            h_all.reshape(r * n, hdim), idx, dst, norm, n, hdim, nc, ns
        )
        h = _tc_layer_c(agg2, root, h, ln_g[l], ln_b[l])

    batch3d = batch.reshape(n // _BLK, 1, _BLK)
    return _tc_pool(h, batch3d, g)


# speedup vs baseline: 34.9269x; 2.3749x over previous
import functools

import jax
import jax.numpy as jnp
from jax import lax
from jax.experimental import pallas as pl
from jax.experimental.pallas import tpu as pltpu
from jax.experimental.pallas import tpu_sc as plsc


_BLK = 1000


def _mm(a, b):
    return jnp.dot(a, b, preferred_element_type=jnp.float32)


def _emit_a(hv, wrel_ref, wroot_ref, bc_ref, hall_ref, root_ref):
    for i in range(wrel_ref.shape[0]):
        hall_ref[i] = _mm(hv, wrel_ref[i])
    root_ref[...] = _mm(hv, wroot_ref[...]) + bc_ref[...]


def _pa_body(x_ref, win_ref, bin_ref, wrel_ref, wroot_ref, bc_ref,
             h_ref, hall_ref, root_ref):
    h0 = _mm(x_ref[...], win_ref[...]) + bin_ref[...]
    h_ref[...] = h0
    _emit_a(h0, wrel_ref, wroot_ref, bc_ref, hall_ref, root_ref)


def _tc_pa(x, win, bin_, wrel, wroot, bc):
    n, d = x.shape
    h = win.shape[1]
    r = wrel.shape[0]
    return pl.pallas_call(
        _pa_body,
        grid=(n // _BLK,),
        in_specs=[
            pl.BlockSpec((_BLK, d), lambda i: (i, 0)),
            pl.BlockSpec((d, h), lambda i: (0, 0)),
            pl.BlockSpec((1, h), lambda i: (0, 0)),
            pl.BlockSpec((r, h, h), lambda i: (0, 0, 0)),
            pl.BlockSpec((h, h), lambda i: (0, 0)),
            pl.BlockSpec((1, h), lambda i: (0, 0)),
        ],
        out_specs=[
            pl.BlockSpec((_BLK, h), lambda i: (i, 0)),
            pl.BlockSpec((r, _BLK, h), lambda i: (0, i, 0)),
            pl.BlockSpec((_BLK, h), lambda i: (i, 0)),
        ],
        out_shape=[
            jax.ShapeDtypeStruct((n, h), jnp.float32),
            jax.ShapeDtypeStruct((r, n, h), jnp.float32),
            jax.ShapeDtypeStruct((n, h), jnp.float32),
        ],
    )(x, win, bin_.reshape(1, h), wrel, wroot, bc.reshape(1, h))


def _new_h(agg_ref, root_ref, hprev_ref, g_ref, b_ref):
    s = agg_ref[0] + agg_ref[1] + root_ref[...]
    mu = jnp.mean(s, axis=-1, keepdims=True)
    var = jnp.mean((s - mu) ** 2, axis=-1, keepdims=True)
    y = (s - mu) / jnp.sqrt(var + 1e-5) * g_ref[...] + b_ref[...]
    return jnp.maximum(y, 0.0) + hprev_ref[...]


def _ac_body(agg_ref, root_ref, hprev_ref, g_ref, b_ref,
             wrel_ref, wroot_ref, bc_ref, h_ref, hall_ref, rootout_ref):
    hnew = _new_h(agg_ref, root_ref, hprev_ref, g_ref, b_ref)
    h_ref[...] = hnew
    _emit_a(hnew, wrel_ref, wroot_ref, bc_ref, hall_ref, rootout_ref)


def _tc_ac(agg2, root, hx, g, b, wrel, wroot, bc):
    n, h = hx.shape
    r = wrel.shape[0]
    return pl.pallas_call(
        _ac_body,
        grid=(n // _BLK,),
        in_specs=[
            pl.BlockSpec((2, _BLK, h), lambda i: (0, i, 0)),
            pl.BlockSpec((_BLK, h), lambda i: (i, 0)),
            pl.BlockSpec((_BLK, h), lambda i: (i, 0)),
            pl.BlockSpec((1, h), lambda i: (0, 0)),
            pl.BlockSpec((1, h), lambda i: (0, 0)),
            pl.BlockSpec((r, h, h), lambda i: (0, 0, 0)),
            pl.BlockSpec((h, h), lambda i: (0, 0)),
            pl.BlockSpec((1, h), lambda i: (0, 0)),
        ],
        out_specs=[
            pl.BlockSpec((_BLK, h), lambda i: (i, 0)),
            pl.BlockSpec((r, _BLK, h), lambda i: (0, i, 0)),
            pl.BlockSpec((_BLK, h), lambda i: (i, 0)),
        ],
        out_shape=[
            jax.ShapeDtypeStruct((n, h), jnp.float32),
            jax.ShapeDtypeStruct((r, n, h), jnp.float32),
            jax.ShapeDtypeStruct((n, h), jnp.float32),
        ],
    )(agg2, root, hx, g.reshape(1, h), b.reshape(1, h),
      wrel, wroot, bc.reshape(1, h))


def _cpool_body(agg_ref, root_ref, hprev_ref, g_ref, b_ref, batch_ref, o_ref):
    hnew = _new_h(agg_ref, root_ref, hprev_ref, g_ref, b_ref)
    ng = o_ref.shape[0]
    blk = hnew.shape[0]

    @pl.when(pl.program_id(0) == 0)
    def _():
        o_ref[...] = jnp.zeros_like(o_ref)

    bvec = batch_ref[0]
    onehot = (
        bvec == lax.broadcasted_iota(jnp.int32, (ng, blk), 0)
    ).astype(jnp.float32)
    o_ref[...] += lax.dot_general(
        onehot, hnew, (((1,), (0,)), ((), ())),
        preferred_element_type=jnp.float32,
    )


def _tc_cpool(agg2, root, hx, g, b, batch3d, ng):
    n, h = hx.shape
    return pl.pallas_call(
        _cpool_body,
        grid=(n // _BLK,),
        in_specs=[
            pl.BlockSpec((2, _BLK, h), lambda i: (0, i, 0)),
            pl.BlockSpec((_BLK, h), lambda i: (i, 0)),
            pl.BlockSpec((_BLK, h), lambda i: (i, 0)),
            pl.BlockSpec((1, h), lambda i: (0, 0)),
            pl.BlockSpec((1, h), lambda i: (0, 0)),
            pl.BlockSpec((1, 1, _BLK), lambda i: (i, 0, 0)),
        ],
        out_specs=pl.BlockSpec((ng, h), lambda i: (0, 0)),
        out_shape=jax.ShapeDtypeStruct((ng, h), jnp.float32),
    )(agg2, root, hx, g.reshape(1, h), b.reshape(1, h), batch3d)


def _inv_body(cnt_ref, o_ref):
    c = cnt_ref[0] + cnt_ref[1]
    o_ref[...] = jnp.where(c > 0, 1.0 / jnp.maximum(c, 1.0), 0.0)


def _tc_inv(cnt2_3d):
    _, rows, cols = cnt2_3d.shape
    return pl.pallas_call(
        _inv_body,
        grid=(1,),
        in_specs=[pl.BlockSpec((2, rows, cols), lambda i: (0, 0, 0))],
        out_specs=pl.BlockSpec((rows, cols), lambda i: (0, 0)),
        out_shape=jax.ShapeDtypeStruct((rows, cols), jnp.float32),
    )(cnt2_3d)


_LANES = 16


def _zero_fill(ref, nelem):
    z = jnp.zeros((_LANES,), jnp.float32)

    def body(i, _):
        ref[pl.ds(i * _LANES, _LANES)] = z
        return 0

    lax.fori_loop(0, nelem // _LANES, body, 0, unroll=4)


def _sc_count(key16, nk, nc, ns):
    e = key16.shape[0]
    nw = nc * ns
    per_w = e // nw
    ck = 2000
    n_chunks = per_w // ck
    per_tile = nk // ns
    mesh = plsc.VectorSubcoreMesh(core_axis_name="c", subcore_axis_name="s")

    @functools.partial(
        pl.kernel,
        out_type=jax.ShapeDtypeStruct((2 * nk,), jnp.float32),
        mesh=mesh,
        scratch_types=[
            pltpu.VMEM((ck,), jnp.int32),
            pltpu.VMEM((ck,), jnp.float32),
            pltpu.VMEM((per_tile,), jnp.float32),
            pltpu.VMEM_SHARED((nk,), jnp.float32),
            pltpu.SemaphoreType.DMA,
        ],
    )
    def k(key_hbm, out_hbm, key_v, ones_v, zb, cnt_sp, sem):
        cid = lax.axis_index("c")
        sid = lax.axis_index("s")
        wid = sid * nc + cid

        o = jnp.ones((_LANES,), jnp.float32)

        def fill_ones(i, _):
            ones_v[pl.ds(i * _LANES, _LANES)] = o
            return 0

        lax.fori_loop(0, ck // _LANES, fill_ones, 0, unroll=4)

        _zero_fill(zb, per_tile)
        pltpu.sync_copy(zb, cnt_sp.at[pl.ds(sid * per_tile, per_tile)])
        plsc.subcore_barrier()

        def chunk(i, _):
            base = wid * per_w + i * ck
            pltpu.sync_copy(key_hbm.at[pl.ds(base, ck)], key_v)
            pltpu.sync_copy(ones_v, cnt_sp.at[key_v], add=True)
            return 0

        lax.fori_loop(0, n_chunks, chunk, 0)
        plsc.subcore_barrier()

        pltpu.sync_copy(cnt_sp.at[pl.ds(sid * per_tile, per_tile)], zb)
        pltpu.sync_copy(zb, out_hbm.at[pl.ds(cid * nk + sid * per_tile, per_tile)])

    return k(key16).reshape(2, nk)


def _sc_norm_gather(invflat, key16, nc, ns):
    e = key16.shape[0]
    nw = nc * ns
    per_w = e // nw
    ck = 2000
    n_chunks = per_w // ck
    mesh = plsc.VectorSubcoreMesh(core_axis_name="c", subcore_axis_name="s")

    @functools.partial(
        pl.kernel,
        out_type=jax.ShapeDtypeStruct((e,), jnp.float32),
        mesh=mesh,
        scratch_types=[
            pltpu.VMEM((ck,), jnp.int32),
            pltpu.VMEM((ck,), jnp.float32),
            pltpu.SemaphoreType.DMA,
        ],
    )
    def k(inv_hbm, key_hbm, out_hbm, key_v, nv, sem):
        cid = lax.axis_index("c")
        sid = lax.axis_index("s")
        wid = sid * nc + cid

        def chunk(i, _):
            base = wid * per_w + i * ck
            pltpu.sync_copy(key_hbm.at[pl.ds(base, ck)], key_v)
            pltpu.async_copy(inv_hbm.at[key_v], nv, sem).wait()
            pltpu.sync_copy(nv, out_hbm.at[pl.ds(base, ck)])
            return 0

        lax.fori_loop(0, n_chunks, chunk, 0)

    return k(invflat, key16)


def _sc_layer_agg(hall_flat, idx, dst, norm, n, h, nc, ns):
    e = idx.shape[0]
    nw = nc * ns
    per_w = e // nw
    ek = 80
    n_chunks = per_w // ek
    n_pairs = (n_chunks - 1) // 2
    zrows = ek
    n_blocks = n // zrows
    mesh = plsc.VectorSubcoreMesh(core_axis_name="c", subcore_axis_name="s")

    idx2 = idx.reshape(nw, per_w)
    dst2 = dst.reshape(nw, per_w)
    norm2 = norm.reshape(nw, per_w)

    @functools.partial(
        pl.kernel,
        out_type=jax.ShapeDtypeStruct((2, n, h), jnp.float32),
        mesh=mesh,
        scratch_types=[
            pltpu.VMEM((per_w,), jnp.int32),
            pltpu.VMEM((per_w,), jnp.int32),
            pltpu.VMEM((per_w,), jnp.float32),
            pltpu.VMEM((2, ek, h), jnp.float32),
            pltpu.VMEM((ek,), jnp.int32),
            pltpu.VMEM((ek,), jnp.int32),
            pltpu.VMEM_SHARED((n, h), jnp.float32),
            pltpu.SemaphoreType.DMA,
            pltpu.SemaphoreType.DMA,
            pltpu.SemaphoreType.DMA,
            pltpu.SemaphoreType.DMA,
        ],
    )
    def k(hall_hbm, idx_hbm, dst_hbm, norm_hbm, out_hbm,
          idx_v, dst_v, norm_v, rows_v, dstc0, dstc1, agg_sp,
          gsem0, gsem1, ssem0, ssem1):
        cid = lax.axis_index("c")
        sid = lax.axis_index("s")
        wid = sid * nc + cid
        gsems = (gsem0, gsem1)
        ssems = (ssem0, ssem1)
        dstcs = (dstc0, dstc1)

        pltpu.async_copy(idx_hbm.at[wid], idx_v, gsem0)
        pltpu.async_copy(dst_hbm.at[wid], dst_v, gsem0)
        pltpu.async_copy(norm_hbm.at[wid], norm_v, gsem0)

        z = jnp.zeros((_LANES,), jnp.float32)

        def zfill(i, _):
            r = i // (h // _LANES)
            c = i % (h // _LANES)
            rows_v[0, r, pl.ds(c * _LANES, _LANES)] = z
            return 0

        lax.fori_loop(0, zrows * (h // _LANES), zfill, 0, unroll=4)
        zsrc = rows_v.at[0]
        for j in range((n_blocks + ns - 1) // ns):
            blk = j * ns + sid

            @pl.when(blk < n_blocks)
            def _():
                pltpu.sync_copy(zsrc, agg_sp.at[pl.ds(blk * zrows, zrows)])
        pltpu.make_async_copy(idx_hbm.at[wid], idx_v, gsem0).wait()
        pltpu.make_async_copy(dst_hbm.at[wid], dst_v, gsem0).wait()
        pltpu.make_async_copy(norm_hbm.at[wid], norm_v, gsem0).wait()
        plsc.subcore_barrier()

        def g_start(c, b):
            pltpu.async_copy(
                hall_hbm.at[idx_v.at[pl.ds(c * ek, ek)]], rows_v.at[b], gsems[b]
            )

        def g_wait(b):
            pltpu.make_async_copy(
                hall_hbm.at[idx_v.at[pl.ds(0, ek)]], rows_v.at[b], gsems[b]
            ).wait()

        def scale(c, b):
            def grp(gi, _):
                nv16 = norm_v[pl.ds(c * ek + gi * _LANES, _LANES)]
                for l in range(_LANES):
                    ei = gi * _LANES + l
                    nb = jnp.broadcast_to(nv16[l], (_LANES,))
                    for cc in range(h // _LANES):
                        sl = pl.ds(cc * _LANES, _LANES)
                        rows_v[b, ei, sl] = rows_v[b, ei, sl] * nb
                return 0

            lax.fori_loop(0, ek // _LANES, grp, 0)

        def s_start(c, b):
            dstc = dstcs[b]
            for g in range(ek // _LANES):
                sl = pl.ds(g * _LANES, _LANES)
                dstc[sl] = dst_v[pl.ds(c * ek + g * _LANES, _LANES)]
            pltpu.async_copy(rows_v.at[b], agg_sp.at[dstc], ssems[b], add=True)

        def s_wait(b):
            pltpu.make_async_copy(
                rows_v.at[b], agg_sp.at[dstcs[b]], ssems[b]
            ).wait()

        g_start(0, 0)

        def pair(j, _):
            c0 = 2 * j
            g_wait(0)

            @pl.when(j > 0)
            def _():
                s_wait(1)

            g_start(c0 + 1, 1)
            scale(c0, 0)
            s_start(c0, 0)
            g_wait(1)
            s_wait(0)
            g_start(c0 + 2, 0)
            scale(c0 + 1, 1)
            s_start(c0 + 1, 1)
            return 0

        lax.fori_loop(0, n_pairs, pair, 0)
        g_wait(0)
        s_wait(1)
        scale(n_chunks - 1, 0)
        s_start(n_chunks - 1, 0)
        s_wait(0)
        plsc.subcore_barrier()

        njj = (n_blocks + ns - 1) // ns
        for j in range(njj):
            blk = j * ns + sid
            b = j % 2

            @pl.when(blk < n_blocks)
            def _():
                if j >= 2:
                    pltpu.make_async_copy(
                        rows_v.at[b],
                        out_hbm.at[cid, pl.ds((j - 2) * ns * zrows, zrows)],
                        gsems[b],
                    ).wait()
                r0 = blk * zrows
                pltpu.sync_copy(agg_sp.at[pl.ds(r0, zrows)], rows_v.at[b])
                pltpu.async_copy(
                    rows_v.at[b], out_hbm.at[cid, pl.ds(r0, zrows)], gsems[b]
                )
        for j in range(max(njj - 2, 0), njj):
            blk = j * ns + sid
            b = j % 2

            @pl.when(blk < n_blocks)
            def _():
                pltpu.make_async_copy(
                    rows_v.at[b],
                    out_hbm.at[cid, pl.ds(blk * zrows, zrows)],
                    gsems[b],
                ).wait()

    return k(hall_flat, idx2, dst2, norm2)


def kernel(x, edge_index, edge_type, batch, W_in, b_in, W_rel, W_root,
           b_conv, ln_g, ln_b):
    n, d = x.shape
    hdim = W_in.shape[1]
    nlayers, r = W_rel.shape[0], W_rel.shape[1]
    g = 16
    info = plsc.get_sparse_core_info()
    nc, ns = info.num_cores, info.num_subcores

    src = edge_index[0]
    dst = edge_index[1]
    idx = edge_type * n + src
    key16 = dst * 16 + edge_type

    nk = n * 16
    cnt2 = _sc_count(key16, nk, nc, ns)
    inv = _tc_inv(cnt2.reshape(2, nk // 128, 128))
    norm = _sc_norm_gather(inv.reshape(nk), key16, nc, ns)

    batch3d = batch.reshape(n // _BLK, 1, _BLK)
    h, h_all, root = _tc_pa(x, W_in, b_in, W_rel[0], W_root[0], b_conv[0])
    for l in range(nlayers):
        agg2 = _sc_layer_agg(
            h_all.reshape(r * n, hdim), idx, dst, norm, n, hdim, nc, ns
        )
        if l + 1 < nlayers:
            h, h_all, root = _tc_ac(
                agg2, root, h, ln_g[l], ln_b[l],
                W_rel[l + 1], W_root[l + 1], b_conv[l + 1],
            )
    return _tc_cpool(
        agg2, root, h, ln_g[nlayers - 1], ln_b[nlayers - 1], batch3d, g
    )
